# baseline (speedup 1.0000x reference)
import functools

import jax
import jax.numpy as jnp
from jax import lax
from jax.experimental import pallas as pl
from jax.experimental.pallas import tpu as pltpu
from jax.experimental.pallas import tpu_sc as plsc

_H = 4
_N = 2048
_D = 256
_K = 32
_B = 256

_NW = 32
_RPW = (_H * _N) // _NW
_RBLK = 16


def _body_a(e1_ref, e2_ref, tau_ref, e_out_ref, z_out_ref):
    for h in range(_H):
        a = e1_ref[h]
        bm = e2_ref[h]
        logits = jnp.dot(a, bm, preferred_element_type=jnp.float32,
                         precision=jax.lax.Precision.DEFAULT)
        s = jnp.maximum(logits, 0.0) / tau_ref[h]
        rowmax = jnp.max(s, axis=1, keepdims=True)
        e = jnp.exp(s - rowmax)
        e_out_ref[h] = e
        z_out_ref[h] = jnp.sum(e, axis=1)


def _sc_body(e_hbm, t_hbm, rows_v, tvec_ref, hist_ref):
    wid = lax.axis_index("s") * 2 + lax.axis_index("c")
    base = wid * _RPW
    lanes = lax.iota(jnp.int32, 16)
    zeros16 = jnp.zeros((16,), jnp.int32)
    ones16 = jnp.full((16,), 1, jnp.int32)
    ninf16 = jnp.full((16,), -jnp.inf, jnp.float32)

    def block_loop(blk, _):
        row0 = base + blk * _RBLK
        pltpu.sync_copy(e_hbm.at[pl.ds(row0, _RBLK)], rows_v)

        def row_loop(r, tv):
            def zero_loop(q, c):
                hist_ref[pl.ds(q * 16, 16)] = zeros16
                return c
            lax.fori_loop(0, 64, zero_loop, 0)

            def hist_loop(c, carry):
                v = rows_v[r, pl.ds(c * 16, 16)]
                b = jnp.clip((v * 2048.0).astype(jnp.int32) - 1024, 0, 1023)
                plsc.addupdate_scatter(hist_ref, [b], ones16)
                return carry
            lax.fori_loop(0, 128, hist_loop, 0)

            def scan_loop(i, carry):
                running, bstar, rank = carry
                jj = 63 - i
                h = hist_ref[pl.ds(jj * 16, 16)]
                stot = jnp.sum(h)
                cum = plsc.cumsum(h)
                above = (running + stot) - cum
                cross = (above < _K) & ((above + h) >= _K)
                cand_b = jnp.where(cross, lanes + jj * 16, -1)
                cand_r = jnp.where(cross, above, -1)
                bstar = jnp.maximum(bstar, jnp.max(cand_b))
                rank = jnp.maximum(rank, jnp.max(cand_r))
                return (running + stot, bstar, rank)

            _, bstar, rank_above = lax.fori_loop(
                0, 64, scan_loop, (jnp.int32(0), jnp.int32(-1), jnp.int32(-1)))

            ub = (bstar.astype(jnp.float32) + 1025.0) * (1.0 / 2048.0)
            ub = jnp.where(bstar >= 1023, jnp.float32(2.0), ub)

            def walk(step, m):
                def inner(c, acc):
                    x = rows_v[r, pl.ds(c * 16, 16)]
                    return jnp.maximum(acc, jnp.where(x < m, x, ninf16))
                acc = lax.fori_loop(0, 128, inner, ninf16)
                return jnp.max(acc)

            t = lax.fori_loop(0, _K - rank_above, walk, ub)
            return jnp.where(lanes == r, t, tv)

        tv = lax.fori_loop(0, _RBLK, row_loop,
                           jnp.zeros((16,), jnp.float32))
        tvec_ref[...] = tv
        pltpu.sync_copy(tvec_ref, t_hbm.at[pl.ds(row0, _RBLK)])
        return 0

    lax.fori_loop(0, _RPW // _RBLK, block_loop, 0)


def _body_b(e_ref, z_ref, t_ref, w1_ref, b1_ref, w2_ref, b2_ref,
            final_ref, adjs_ref):
    e_all = e_ref[...]
    z = z_ref[...][:, :, None]
    t = t_ref[...][:, :, None]
    masked = jnp.where(e_all >= t, e_all, 0.0)
    st = jnp.sum(masked, axis=2, keepdims=True)
    adj = masked / (st + 1e-8 * z)
    adjs_ref[...] = adj
    adj_heads = [adj[h] for h in range(_H)]

    adj_mean = (adj_heads[0] + adj_heads[1] + adj_heads[2] + adj_heads[3]) \
        * (1.0 / _H)

    ew = None
    for k in range(2 * _H):
        acc = adj_heads[0] * w1_ref[k, 0]
        for h in range(1, _H):
            acc = acc + adj_heads[h] * w1_ref[k, h]
        hk = jnp.maximum(acc + b1_ref[k], 0.0)
        contrib = hk * w2_ref[0, k]
        ew = contrib if ew is None else ew + contrib
    sig = 1.0 / (1.0 + jnp.exp(-(ew + b2_ref[0])))
    final_ref[...] = sig * adj_mean


def kernel(emb1, emb2, temperature, W1, b1, W2, b2):
    smem = pl.BlockSpec(memory_space=pltpu.MemorySpace.SMEM)
    grid = (_N // _B,)

    e_arr, z_arr = pl.pallas_call(
        _body_a,
        grid=grid,
        in_specs=[
            pl.BlockSpec((_H, _B, _D), lambda i: (0, i, 0)),
            pl.BlockSpec((_H, _D, _N), lambda i: (0, 0, 0)),
            smem,
        ],
        out_specs=[
            pl.BlockSpec((_H, _B, _N), lambda i: (0, i, 0)),
            pl.BlockSpec((_H, _B), lambda i: (0, i)),
        ],
        out_shape=[
            jax.ShapeDtypeStruct((_H, _N, _N), jnp.float32),
            jax.ShapeDtypeStruct((_H, _N), jnp.float32),
        ],
    )(emb1, emb2, temperature)

    mesh = plsc.VectorSubcoreMesh(core_axis_name="c", subcore_axis_name="s")
    sc_thresh = functools.partial(
        pl.kernel, mesh=mesh,
        compiler_params=pltpu.CompilerParams(needs_layout_passes=False),
        out_type=jax.ShapeDtypeStruct((_H * _N,), jnp.float32),
        scratch_types=[
            pltpu.VMEM((_RBLK, _N), jnp.float32),
            pltpu.VMEM((16,), jnp.float32),
            pltpu.VMEM((1024,), jnp.int32),
        ],
    )(_sc_body)
    t_flat = sc_thresh(e_arr.reshape(_H * _N, _N))
    t_arr = t_flat.reshape(_H, _N)

    final_adj, adjs = pl.pallas_call(
        _body_b,
        grid=grid,
        in_specs=[
            pl.BlockSpec((_H, _B, _N), lambda i: (0, i, 0)),
            pl.BlockSpec((_H, _B), lambda i: (0, i)),
            pl.BlockSpec((_H, _B), lambda i: (0, i)),
            smem, smem, smem, smem,
        ],
        out_specs=[
            pl.BlockSpec((_B, _N), lambda i: (i, 0)),
            pl.BlockSpec((_H, _B, _N), lambda i: (0, i, 0)),
        ],
        out_shape=[
            jax.ShapeDtypeStruct((_N, _N), jnp.float32),
            jax.ShapeDtypeStruct((_H, _N, _N), jnp.float32),
        ],
    )(e_arr, z_arr, t_arr, W1, b1, W2, b2)
    return (final_adj, adjs)


# speedup vs baseline: 3.1410x; 3.1410x over previous
import jax
import jax.numpy as jnp
from jax.experimental import pallas as pl
from jax.experimental.pallas import tpu as pltpu

_H = 4
_N = 2048
_D = 256
_K = 32
_B = 256


def _body(e1_ref, e2_ref, tau_ref, w1_ref, b1_ref, w2_ref, b2_ref,
          final_ref, adjs_ref):
    es = []
    for h in range(_H):
        a = e1_ref[h]
        bm = e2_ref[h]
        logits = jnp.dot(a, bm, preferred_element_type=jnp.float32,
                         precision=jax.lax.Precision.DEFAULT)
        s = jnp.maximum(logits, 0.0) / tau_ref[h]
        rowmax = jnp.max(s, axis=1, keepdims=True)
        es.append(jnp.exp(s - rowmax))
    e_all = jnp.stack(es, axis=0)
    z = jnp.sum(e_all, axis=2, keepdims=True)

    def _next_below(_, m):
        return jnp.max(jnp.where(e_all < m, e_all, -jnp.inf), axis=2,
                       keepdims=True)

    m0 = jnp.max(e_all, axis=2, keepdims=True)
    t = jax.lax.fori_loop(0, _K - 1, _next_below, m0, unroll=4)

    for _ in range(4):
        nu = jnp.min(jnp.where(e_all > t, e_all, jnp.inf), axis=2,
                     keepdims=True)
        c_nu = jnp.sum(jnp.where(e_all >= nu, 1.0, 0.0), axis=2,
                       keepdims=True)
        t = jnp.where(c_nu >= _K, nu, t)

    masked = jnp.where(e_all >= t, e_all, 0.0)
    st = jnp.sum(masked, axis=2, keepdims=True)
    adj = masked / (st + 1e-8 * z)
    adjs_ref[...] = adj
    adj_heads = [adj[h] for h in range(_H)]

    adj_mean = (adj_heads[0] + adj_heads[1] + adj_heads[2] + adj_heads[3]) \
        * (1.0 / _H)

    ew = None
    for k in range(2 * _H):
        acc = adj_heads[0] * w1_ref[k, 0]
        for h in range(1, _H):
            acc = acc + adj_heads[h] * w1_ref[k, h]
        hk = jnp.maximum(acc + b1_ref[k], 0.0)
        contrib = hk * w2_ref[0, k]
        ew = contrib if ew is None else ew + contrib
    sig = 1.0 / (1.0 + jnp.exp(-(ew + b2_ref[0])))
    final_ref[...] = sig * adj_mean


def kernel(emb1, emb2, temperature, W1, b1, W2, b2):
    smem = pl.BlockSpec(memory_space=pltpu.MemorySpace.SMEM)
    grid = (_N // _B,)
    final_adj, adjs = pl.pallas_call(
        _body,
        grid=grid,
        in_specs=[
            pl.BlockSpec((_H, _B, _D), lambda i: (0, i, 0)),
            pl.BlockSpec((_H, _D, _N), lambda i: (0, 0, 0)),
            smem, smem, smem, smem, smem,
        ],
        out_specs=[
            pl.BlockSpec((_B, _N), lambda i: (i, 0)),
            pl.BlockSpec((_H, _B, _N), lambda i: (0, i, 0)),
        ],
        out_shape=[
            jax.ShapeDtypeStruct((_N, _N), jnp.float32),
            jax.ShapeDtypeStruct((_H, _N, _N), jnp.float32),
        ],
    )(emb1, emb2, temperature, W1, b1, W2, b2)
    return (final_adj, adjs)


# speedup vs baseline: 3.1729x; 1.0102x over previous
import jax
import jax.numpy as jnp
from jax.experimental import pallas as pl
from jax.experimental.pallas import tpu as pltpu

_H = 4
_N = 2048
_D = 256
_K = 32
_B = 256


def _body(e1_ref, e2_ref, tau_ref, w1_ref, b1_ref, w2_ref, b2_ref,
          final_ref, adjs_ref):
    es = []
    for h in range(_H):
        a = e1_ref[h]
        bm = e2_ref[h]
        logits = jnp.dot(a, bm, preferred_element_type=jnp.float32,
                         precision=jax.lax.Precision.DEFAULT)
        s = jnp.maximum(logits, 0.0) / tau_ref[h]
        rowmax = jnp.max(s, axis=1, keepdims=True)
        es.append(jnp.exp(s - rowmax))
    e_all = jnp.stack(es, axis=0)
    z = jnp.sum(e_all, axis=2, keepdims=True)

    def _next_below(_, m):
        return jnp.max(jnp.where(e_all < m, e_all, -jnp.inf), axis=2,
                       keepdims=True)

    m0 = jnp.max(e_all, axis=2, keepdims=True)
    t = jax.lax.fori_loop(0, _K - 1, _next_below, m0, unroll=8)

    for _ in range(4):
        nu = jnp.min(jnp.where(e_all > t, e_all, jnp.inf), axis=2,
                     keepdims=True)
        c_nu = jnp.sum(jnp.where(e_all >= nu, 1.0, 0.0), axis=2,
                       keepdims=True)
        t = jnp.where(c_nu >= _K, nu, t)

    masked = jnp.where(e_all >= t, e_all, 0.0)
    st = jnp.sum(masked, axis=2, keepdims=True)
    adj = masked / (st + 1e-8 * z)
    adjs_ref[...] = adj
    adj_heads = [adj[h] for h in range(_H)]

    adj_mean = (adj_heads[0] + adj_heads[1] + adj_heads[2] + adj_heads[3]) \
        * (1.0 / _H)

    ew = None
    for k in range(2 * _H):
        acc = adj_heads[0] * w1_ref[k, 0]
        for h in range(1, _H):
            acc = acc + adj_heads[h] * w1_ref[k, h]
        hk = jnp.maximum(acc + b1_ref[k], 0.0)
        contrib = hk * w2_ref[0, k]
        ew = contrib if ew is None else ew + contrib
    sig = 1.0 / (1.0 + jnp.exp(-(ew + b2_ref[0])))
    final_ref[...] = sig * adj_mean


def kernel(emb1, emb2, temperature, W1, b1, W2, b2):
    smem = pl.BlockSpec(memory_space=pltpu.MemorySpace.SMEM)
    grid = (_N // _B,)
    final_adj, adjs = pl.pallas_call(
        _body,
        grid=grid,
        in_specs=[
            pl.BlockSpec((_H, _B, _D), lambda i: (0, i, 0)),
            pl.BlockSpec((_H, _D, _N), lambda i: (0, 0, 0)),
            smem, smem, smem, smem, smem,
        ],
        out_specs=[
            pl.BlockSpec((_B, _N), lambda i: (i, 0)),
            pl.BlockSpec((_H, _B, _N), lambda i: (0, i, 0)),
        ],
        out_shape=[
            jax.ShapeDtypeStruct((_N, _N), jnp.float32),
            jax.ShapeDtypeStruct((_H, _N, _N), jnp.float32),
        ],
    )(emb1, emb2, temperature, W1, b1, W2, b2)
    return (final_adj, adjs)
